# baseline (speedup 1.0000x reference)
import functools
import math
import random

import numpy as np
import jax
import jax.numpy as jnp
from jax import lax
from jax.experimental import pallas as pl
from jax.experimental.pallas import tpu as pltpu
from jax.experimental.pallas import tpu_sc as plsc

B, S, D = 4, 4096, 1024
R = B * S

NC, NS = 2, 16
NW = NC * NS
RPW = R // NW
CH = 32
NCH = RPW // CH
NBUF = 3


def _static_plan():
    np.random.seed(0)
    random.seed(0)
    alpha = 1.0
    num_seg = 3

    x_len = S
    token_len = math.ceil(x_len / (num_seg - 1))
    sx = int(np.random.randint(int(token_len / 4), int(token_len * 3 / 4)))
    seq_src = []
    for ii in random.sample(range(num_seg), num_seg):
        b1 = int(np.clip(sx + token_len * (ii - 1), 0, x_len))
        b2 = int(np.clip(sx + token_len * ii, 0, x_len))
        seq_src.append(np.arange(b1, b2))
    seq_src = np.concatenate(seq_src)

    lam = float(np.random.beta(alpha, alpha))
    index = np.random.permutation(B)
    cut_len = int(x_len * (1.0 - lam))
    cx = int(np.random.randint(x_len))
    bbx1 = int(np.clip(cx - cut_len // 2, 0, x_len))
    bbx2 = int(np.clip(cx + cut_len // 2, 0, x_len))
    lam_out = 1.0 - (bbx2 - bbx1) / x_len

    src = np.empty((B, S), np.int32)
    for b in range(B):
        src[b, :] = b * S + seq_src
        src[b, bbx1:bbx2] = index[b] * S + seq_src[bbx1:bbx2]
    return src.reshape(-1), np.float32(lam_out), index


_SRC_ROWS, _LAM, _INDEX = _static_plan()
_IDX_NP = np.ascontiguousarray(_SRC_ROWS.reshape(NW, NCH, CH))

@functools.lru_cache(maxsize=None)
def _build_gather():
    mesh = plsc.VectorSubcoreMesh(
        core_axis_name="c", subcore_axis_name="s",
        num_cores=NC, num_subcores=NS)

    @functools.partial(
        pl.kernel,
        out_type=jax.ShapeDtypeStruct((R, D), jnp.float32),
        mesh=mesh,
        scratch_types=(
            [pltpu.VMEM((NCH, CH), jnp.int32)]
            + [pltpu.VMEM((CH, D), jnp.float32) for _ in range(NBUF)]
            + [pltpu.SemaphoreType.DMA for _ in range(2 * NBUF)]
        ),
        compiler_params=pltpu.CompilerParams(use_tc_tiling_on_sc=False),
    )
    def _gather_rows(x_hbm, idx_hbm, out_hbm, idx_v, *scratch):
        bufs = scratch[:NBUF]
        gsems = scratch[NBUF:2 * NBUF]
        ssems = scratch[2 * NBUF:]
        wid = lax.axis_index("s") * NC + lax.axis_index("c")
        base = wid * RPW
        pltpu.sync_copy(idx_hbm.at[wid], idx_v)

        def gather(c):
            k = c % NBUF
            return pltpu.async_copy(x_hbm.at[idx_v.at[c]], bufs[k], gsems[k])

        def scatter(c):
            k = c % NBUF
            return pltpu.async_copy(
                bufs[k], out_hbm.at[pl.ds(base + c * CH, CH)], ssems[k])

        h_g = [None] * NBUF
        h_s = [None] * NBUF
        for c in range(min(NBUF - 1, NCH)):
            h_g[c % NBUF] = gather(c)
        for c in range(NCH):
            g = c + NBUF - 1
            if g < NCH:
                k = g % NBUF
                if h_s[k] is not None:
                    h_s[k].wait()
                h_g[k] = gather(g)
            h_g[c % NBUF].wait()
            h_s[c % NBUF] = scatter(c)
        for k in range(NBUF):
            if h_s[k] is not None:
                h_s[k].wait()

    return _gather_rows


def kernel(x):
    out2d = _build_gather()(x.reshape(R, D), jnp.asarray(_IDX_NP))
    mixed_x = out2d.reshape(B, S, D)
    lam = jnp.float32(_LAM)
    index = jnp.asarray(_INDEX, dtype=jnp.int64)
    return (mixed_x, lam, index)


# speedup vs baseline: 2.5756x; 2.5756x over previous
import functools
import math
import random

import numpy as np
import jax
import jax.numpy as jnp
from jax import lax
from jax.experimental import pallas as pl
from jax.experimental.pallas import tpu as pltpu
from jax.experimental.pallas import tpu_sc as plsc

B, S, D = 4, 4096, 1024
R = B * S
TR = R // 8

NC, NS = 2, 16
NW = NC * NS
TPW = TR // NW
CT = 4
NCH = TPW // CT
NBUF = 3


def _static_plan():
    np.random.seed(0)
    random.seed(0)
    alpha = 1.0
    num_seg = 3

    x_len = S
    token_len = math.ceil(x_len / (num_seg - 1))
    sx = int(np.random.randint(int(token_len / 4), int(token_len * 3 / 4)))
    seq_src = []
    for ii in random.sample(range(num_seg), num_seg):
        b1 = int(np.clip(sx + token_len * (ii - 1), 0, x_len))
        b2 = int(np.clip(sx + token_len * ii, 0, x_len))
        seq_src.append(np.arange(b1, b2))
    seq_src = np.concatenate(seq_src)

    lam = float(np.random.beta(alpha, alpha))
    index = np.random.permutation(B)
    cut_len = int(x_len * (1.0 - lam))
    cx = int(np.random.randint(x_len))
    bbx1 = int(np.clip(cx - cut_len // 2, 0, x_len))
    bbx2 = int(np.clip(cx + cut_len // 2, 0, x_len))
    lam_out = 1.0 - (bbx2 - bbx1) / x_len

    src = np.empty((B, S), np.int32)
    for b in range(B):
        src[b, :] = b * S + seq_src
        src[b, bbx1:bbx2] = index[b] * S + seq_src[bbx1:bbx2]
    return src.reshape(-1), np.float32(lam_out), index


_SRC_ROWS, _LAM, _INDEX = _static_plan()


_ROT_BND = 2900
_ROT_LO, _ROT_HI = 1196, -2900
_SW_LO, _SW_HI = 3231, 3885
_DELTA = {0: 12288, 3: -12288}


def _g_formula(d):
    seq = d % S
    rot = _ROT_LO if seq < _ROT_BND else _ROT_HI
    delta = _DELTA.get(d // S, 0) if _SW_LO <= seq < _SW_HI else 0
    g = (d + rot + delta) // 8
    return min(g, TR - CT)


def _fixup_plan():
    fix = {}
    for w in range(NW):
        for c in range(NCH):
            a0 = w * TPW + c * CT
            rows = []
            for h in (0, 1):
                g = _g_formula(8 * a0 + 4 * h)
                for i in range(CT):
                    for j in range(4):
                        ext = 8 * (g + i) + 4 * (1 - h) + j
                        true = int(_SRC_ROWS[8 * (a0 + i) + 4 * h + j])
                        if ext != true:
                            rows.append((i, 4 * h + j, true // 8, true % 8))
            if rows:
                fix[(w, c)] = rows
    return fix


_FIXUPS = _fixup_plan()


@functools.lru_cache(maxsize=None)
def _build_copy():
    mesh = plsc.VectorSubcoreMesh(
        core_axis_name="c", subcore_axis_name="s",
        num_cores=NC, num_subcores=NS)

    @functools.partial(
        pl.kernel,
        out_type=jax.ShapeDtypeStruct((TR, 8, D), jnp.float32),
        mesh=mesh,
        scratch_types=(
            [pltpu.VMEM((CT, 8, D), jnp.float32) for _ in range(NBUF)]
            + [pltpu.SemaphoreType.DMA for _ in range(2 * NBUF)]
        ),
    )
    def _copy_halftiles(x_hbm, out_hbm, *scratch):
        bufs = scratch[:NBUF]
        gsems = scratch[NBUF:2 * NBUF]
        ssems = scratch[2 * NBUF:]
        wid = lax.axis_index("s") * NC + lax.axis_index("c")
        a_base = wid * TPW

        def src_g(c, h):
            d = 8 * (a_base + c * CT) + 4 * h
            seq = lax.rem(d, S)
            rot = jnp.where(seq < _ROT_BND, _ROT_LO, _ROT_HI)
            b = lax.div(d, S)
            delta = jnp.where(b == 0, _DELTA[0],
                              jnp.where(b == 3, _DELTA[3], 0))
            inswap = jnp.logical_and(seq >= _SW_LO, seq < _SW_HI)
            g = lax.div(d + rot + jnp.where(inswap, delta, 0), 8)
            return jnp.minimum(g, TR - CT)

        def gather(c):
            k = c % NBUF
            hs = []
            for h in (0, 1):
                g = src_g(c, h)
                hs.append(pltpu.async_copy(
                    x_hbm.at[pl.ds(g, CT), pl.ds(4 * (1 - h), 4), :],
                    bufs[k].at[:, pl.ds(4 * h, 4), :],
                    gsems[k]))
            return hs

        def scatter(c):
            k = c % NBUF
            return pltpu.async_copy(
                bufs[k], out_hbm.at[pl.ds(a_base + c * CT, CT)], ssems[k])

        def apply_fixups(c):
            k = c % NBUF
            for w in range(NW):
                rows = _FIXUPS.get((w, c))
                if not rows:
                    continue

                @pl.when(wid == w)
                def _(rows=rows, k=k):
                    hs = [pltpu.async_copy(
                        x_hbm.at[pl.ds(sg, 1), pl.ds(ssl, 1), :],
                        bufs[k].at[pl.ds(i, 1), pl.ds(sl, 1), :],
                        gsems[k])
                        for (i, sl, sg, ssl) in rows]
                    for hh in hs:
                        hh.wait()

        h_g = [None] * NBUF
        h_s = [None] * NBUF
        for c in range(min(NBUF - 1, NCH)):
            h_g[c % NBUF] = gather(c)
        for c in range(NCH):
            g = c + NBUF - 1
            if g < NCH:
                k = g % NBUF
                if h_s[k] is not None:
                    h_s[k].wait()
                h_g[k] = gather(g)
            for hh in h_g[c % NBUF]:
                hh.wait()
            apply_fixups(c)
            h_s[c % NBUF] = scatter(c)
        for k in range(NBUF):
            if h_s[k] is not None:
                h_s[k].wait()

    return _copy_halftiles


def kernel(x):
    out3 = _build_copy()(x.reshape(TR, 8, D))
    mixed_x = out3.reshape(B, S, D)
    lam = jnp.float32(_LAM)
    index = jnp.asarray(_INDEX, dtype=jnp.int64)
    return (mixed_x, lam, index)


# speedup vs baseline: 2.7094x; 1.0519x over previous
import functools
import math
import random

import numpy as np
import jax
import jax.numpy as jnp
from jax import lax
from jax.experimental import pallas as pl
from jax.experimental.pallas import tpu as pltpu
from jax.experimental.pallas import tpu_sc as plsc

B, S, D = 4, 4096, 1024
R = B * S

NC, NS = 2, 16
NW = NC * NS
RPW = R // NW
CH = 16
NCH = RPW // CH
NBUF = 6


def _static_plan():
    np.random.seed(0)
    random.seed(0)
    alpha = 1.0
    num_seg = 3

    x_len = S
    token_len = math.ceil(x_len / (num_seg - 1))
    sx = int(np.random.randint(int(token_len / 4), int(token_len * 3 / 4)))
    seq_src = []
    for ii in random.sample(range(num_seg), num_seg):
        b1 = int(np.clip(sx + token_len * (ii - 1), 0, x_len))
        b2 = int(np.clip(sx + token_len * ii, 0, x_len))
        seq_src.append(np.arange(b1, b2))
    seq_src = np.concatenate(seq_src)

    lam = float(np.random.beta(alpha, alpha))
    index = np.random.permutation(B)
    cut_len = int(x_len * (1.0 - lam))
    cx = int(np.random.randint(x_len))
    bbx1 = int(np.clip(cx - cut_len // 2, 0, x_len))
    bbx2 = int(np.clip(cx + cut_len // 2, 0, x_len))
    lam_out = 1.0 - (bbx2 - bbx1) / x_len

    src = np.empty((B, S), np.int32)
    for b in range(B):
        src[b, :] = b * S + seq_src
        src[b, bbx1:bbx2] = index[b] * S + seq_src[bbx1:bbx2]
    return src.reshape(-1), np.float32(lam_out), index


_SRC_ROWS, _LAM, _INDEX = _static_plan()
_IDX_NP = np.ascontiguousarray(_SRC_ROWS.reshape(NW, NCH, CH))

@functools.lru_cache(maxsize=None)
def _build_gather():
    mesh = plsc.VectorSubcoreMesh(
        core_axis_name="c", subcore_axis_name="s",
        num_cores=NC, num_subcores=NS)

    @functools.partial(
        pl.kernel,
        out_type=jax.ShapeDtypeStruct((R, D), jnp.float32),
        mesh=mesh,
        scratch_types=(
            [pltpu.VMEM((NCH, CH), jnp.int32)]
            + [pltpu.VMEM((CH, D), jnp.float32) for _ in range(NBUF)]
            + [pltpu.SemaphoreType.DMA for _ in range(2 * NBUF)]
        ),
    )
    def _gather_rows(x_hbm, idx_hbm, out_hbm, idx_v, *scratch):
        bufs = scratch[:NBUF]
        gsems = scratch[NBUF:2 * NBUF]
        ssems = scratch[2 * NBUF:]
        wid = lax.axis_index("s") * NC + lax.axis_index("c")
        base = wid * RPW
        pltpu.sync_copy(idx_hbm.at[wid], idx_v)

        def gather(c):
            k = c % NBUF
            return pltpu.async_copy(x_hbm.at[idx_v.at[c]], bufs[k], gsems[k])

        def scatter(c):
            k = c % NBUF
            return pltpu.async_copy(
                bufs[k], out_hbm.at[pl.ds(base + c * CH, CH)], ssems[k])

        h_g = [None] * NBUF
        h_s = [None] * NBUF
        for c in range(min(NBUF - 1, NCH)):
            h_g[c % NBUF] = gather(c)
        for c in range(NCH):
            g = c + NBUF - 1
            if g < NCH:
                k = g % NBUF
                if h_s[k] is not None:
                    h_s[k].wait()
                h_g[k] = gather(g)
            h_g[c % NBUF].wait()
            h_s[c % NBUF] = scatter(c)
        for k in range(NBUF):
            if h_s[k] is not None:
                h_s[k].wait()

    return _gather_rows


def kernel(x):
    out2d = _build_gather()(x.reshape(R, D), jnp.asarray(_IDX_NP))
    mixed_x = out2d.reshape(B, S, D)
    lam = jnp.float32(_LAM)
    index = jnp.asarray(_INDEX, dtype=jnp.int64)
    return (mixed_x, lam, index)
